# Initial kernel scaffold; baseline (speedup 1.0000x reference)
import jax
import jax.numpy as jnp
from jax.experimental import pallas as pl


def kernel(embedding, y, idx, memory):
    raise NotImplementedError("write your pallas kernel here")


# speedup vs baseline: 2.7374x; 2.7374x over previous
import jax
import jax.numpy as jnp
from jax import lax
from jax.experimental import pallas as pl
from jax.experimental.pallas import tpu as pltpu
from jax.experimental.pallas import tpu_sc as plsc

B = 1024
D = 128
MEM = 100000
K1 = 1025
T_INV = 1.0 / 0.07
Z_INV = 1.0 / 100000.0

NC = 2
NS = 16
NW = NC * NS
QPW = B // NW

CH = 96
NCHUNK = 11
KPAD = CH * NCHUNK
LANES = 16
NG = CH // LANES


def _group16(rows_ref, e, lane, g):
    acc = jnp.zeros((LANES,), jnp.float32)
    for rr in range(LANES):
        row = g * LANES + rr
        p = rows_ref[row, pl.ds(0, LANES)] * e[0]
        for k in range(1, 8):
            p = p + rows_ref[row, pl.ds(k * LANES, LANES)] * e[k]
        acc = jnp.where(lane == rr, jnp.sum(p), acc)
    return acc


def _sc_body(emb_hbm, idx_hbm, mem_hbm, out_hbm,
             idx_v, rows_v, emb_v, out_v, sem):
    wid = lax.axis_index("s") * NC + lax.axis_index("c")
    lane = lax.iota(jnp.int32, LANES)

    def per_query(q, _):
        b = wid * QPW + q
        pltpu.sync_copy(idx_hbm.at[b], idx_v)
        pltpu.sync_copy(emb_hbm.at[b], emb_v)
        e = [emb_v[pl.ds(k * LANES, LANES)] for k in range(8)]

        def per_chunk(c, _):
            pltpu.async_copy(mem_hbm.at[idx_v.at[c]], rows_v, sem).wait()
            base = c * CH

            def per_group(g, _):
                acc = _group16(rows_v, e, lane, g)
                out_v[pl.ds(base + g * LANES, LANES)] = (
                    jnp.exp(acc * T_INV) * Z_INV)
                return 0

            lax.fori_loop(0, NG, per_group, 0, unroll=False)
            return 0

        lax.fori_loop(0, NCHUNK, per_chunk, 0, unroll=False)
        pltpu.sync_copy(out_v, out_hbm.at[b])
        return 0

    lax.fori_loop(0, QPW, per_query, 0, unroll=False)


@jax.jit
def _nce_sc(emb_r, idx_pad, mem_r):
    mesh = plsc.VectorSubcoreMesh(core_axis_name="c", subcore_axis_name="s",
                                  num_cores=NC, num_subcores=NS)
    f = pl.kernel(
        _sc_body,
        out_type=jax.ShapeDtypeStruct((B, KPAD), jnp.float32),
        mesh=mesh,
        compiler_params=pltpu.CompilerParams(needs_layout_passes=False),
        scratch_types=[
            pltpu.VMEM((NCHUNK, CH), jnp.int32),
            pltpu.VMEM((CH, D), jnp.float32),
            pltpu.VMEM((D,), jnp.float32),
            pltpu.VMEM((KPAD,), jnp.float32),
            pltpu.SemaphoreType.DMA,
        ],
    )
    return f(emb_r, idx_pad, mem_r)


def _round_bf16(x):
    u = lax.bitcast_convert_type(x, jnp.uint32)
    u = u + jnp.uint32(0x7FFF) + ((u >> 16) & jnp.uint32(1))
    return lax.bitcast_convert_type(u & jnp.uint32(0xFFFF0000), jnp.float32)


def kernel(embedding, y, idx, memory):
    del y
    idx_pad = jnp.pad(idx, ((0, 0), (0, KPAD - K1))).reshape(B, NCHUNK, CH)
    out = _nce_sc(_round_bf16(embedding), idx_pad, _round_bf16(memory))
    return out[:, :K1]


# speedup vs baseline: 2.7812x; 1.0160x over previous
import jax
import jax.numpy as jnp
from jax import lax
from jax.experimental import pallas as pl
from jax.experimental.pallas import tpu as pltpu
from jax.experimental.pallas import tpu_sc as plsc

B = 1024
D = 128
MEM = 100000
K1 = 1025
T_INV = 1.0 / 0.07
Z_INV = 1.0 / 100000.0

NC = 2
NS = 16
NW = NC * NS
QPW = B // NW

CH = 96
NCHUNK = 11
KPAD = CH * NCHUNK
LANES = 16
NG = CH // LANES


def _group16(rows_ref, buf, e, lane, g):
    acc = jnp.zeros((LANES,), jnp.float32)
    for rr in range(LANES):
        row = g * LANES + rr
        p = rows_ref[buf, row, pl.ds(0, LANES)] * e[0]
        for k in range(1, 8):
            p = p + rows_ref[buf, row, pl.ds(k * LANES, LANES)] * e[k]
        acc = jnp.where(lane == rr, jnp.sum(p), acc)
    return acc


def _sc_body(emb_hbm, idx_hbm, mem_hbm, out_hbm,
             idx_all, embs_v, rows_v, out_v, sem, osem):
    wid = lax.axis_index("s") * NC + lax.axis_index("c")
    lane = lax.iota(jnp.int32, LANES)
    b0 = wid * QPW

    pltpu.sync_copy(idx_hbm.at[pl.ds(b0, QPW)], idx_all)
    pltpu.sync_copy(emb_hbm.at[pl.ds(b0, QPW)], embs_v)

    def gather_start(q, c, buf):
        pltpu.async_copy(mem_hbm.at[idx_all.at[q, c]], rows_v.at[buf],
                         sem.at[buf])

    def gather_wait(q, c, buf):
        pltpu.make_async_copy(mem_hbm.at[idx_all.at[q, c]], rows_v.at[buf],
                              sem.at[buf]).wait()

    gather_start(0, 0, 0)

    def per_query(q, _):
        obuf = q & 1

        @pl.when(q >= 2)
        def _():
            pltpu.make_async_copy(out_v.at[obuf], out_hbm.at[b0 + q - 2],
                                  osem.at[obuf]).wait()

        e = [embs_v[q, pl.ds(k * LANES, LANES)] for k in range(8)]

        def per_chunk(c, _):
            buf = (q + c) & 1

            last = (q == QPW - 1) & (c == NCHUNK - 1)

            @pl.when(jnp.logical_not(last))
            def _():
                nxt = c == NCHUNK - 1
                nq = jnp.where(nxt, q + 1, q)
                ncc = jnp.where(nxt, 0, c + 1)
                gather_start(nq, ncc, 1 - buf)

            gather_wait(q, c, buf)
            base = c * CH

            def per_group(g, _):
                acc = _group16(rows_v, buf, e, lane, g)
                out_v[obuf, pl.ds(base + g * LANES, LANES)] = (
                    jnp.exp(acc * T_INV) * Z_INV)
                return 0

            lax.fori_loop(0, NG, per_group, 0, unroll=False)
            return 0

        lax.fori_loop(0, NCHUNK, per_chunk, 0, unroll=False)
        pltpu.async_copy(out_v.at[obuf], out_hbm.at[b0 + q], osem.at[obuf])
        return 0

    lax.fori_loop(0, QPW, per_query, 0, unroll=False)

    pltpu.make_async_copy(out_v.at[0], out_hbm.at[b0 + QPW - 2],
                          osem.at[0]).wait()
    pltpu.make_async_copy(out_v.at[1], out_hbm.at[b0 + QPW - 1],
                          osem.at[1]).wait()


@jax.jit
def _nce_sc(emb_r, idx_pad, mem_r):
    mesh = plsc.VectorSubcoreMesh(core_axis_name="c", subcore_axis_name="s",
                                  num_cores=NC, num_subcores=NS)
    f = pl.kernel(
        _sc_body,
        out_type=jax.ShapeDtypeStruct((B, KPAD), jnp.float32),
        mesh=mesh,
        compiler_params=pltpu.CompilerParams(needs_layout_passes=False),
        scratch_types=[
            pltpu.VMEM((QPW, NCHUNK, CH), jnp.int32),
            pltpu.VMEM((QPW, D), jnp.float32),
            pltpu.VMEM((2, CH, D), jnp.float32),
            pltpu.VMEM((2, KPAD), jnp.float32),
            pltpu.SemaphoreType.DMA((2,)),
            pltpu.SemaphoreType.DMA((2,)),
        ],
    )
    return f(emb_r, idx_pad, mem_r)


def _round_bf16(x):
    u = lax.bitcast_convert_type(x, jnp.uint32)
    u = u + jnp.uint32(0x7FFF) + ((u >> 16) & jnp.uint32(1))
    return lax.bitcast_convert_type(u & jnp.uint32(0xFFFF0000), jnp.float32)


def kernel(embedding, y, idx, memory):
    del y
    idx_pad = jnp.pad(idx, ((0, 0), (0, KPAD - K1))).reshape(B, NCHUNK, CH)
    out = _nce_sc(_round_bf16(embedding), idx_pad, _round_bf16(memory))
    return out[:, :K1]
